# Initial kernel scaffold; baseline (speedup 1.0000x reference)
import jax
import jax.numpy as jnp
from jax import lax
from jax.experimental import pallas as pl
from jax.experimental.pallas import tpu as pltpu
from jax.experimental.pallas import tpu_sc as plsc

NC = 2
NS = 16
NW = NC * NS
CH = 128

N = 10000
E = 320000
C = 128
NPAD = 10240
RPS = NPAD // NS
EPAD = ((E + NW * CH - 1) // (NW * CH)) * (NW * CH)
CPW = EPAD // (NW * CH)


def _mesh():
    return plsc.VectorSubcoreMesh(core_axis_name="c", subcore_axis_name="s")


def _hist_body(dstc_hbm, ones_hbm, z16_hbm, out_hbm, dst_v, ones_v, acc):
    c = lax.axis_index("c")
    s = lax.axis_index("s")
    wid = s * NC + c
    r0 = s * RPS
    pltpu.sync_copy(z16_hbm.at[pl.ds(r0, RPS)], acc.at[pl.ds(r0, RPS)])
    pltpu.sync_copy(dstc_hbm.at[pl.ds(wid * CPW, CPW)], dst_v)
    pltpu.sync_copy(ones_hbm, ones_v)
    plsc.subcore_barrier()

    def step(j, carry):
        pltpu.sync_copy(ones_v, acc.at[dst_v.at[j]], add=True)
        return carry

    lax.fori_loop(0, CPW, step, 0)
    plsc.subcore_barrier()
    pltpu.sync_copy(acc.at[pl.ds(r0, RPS)], out_hbm.at[c, pl.ds(r0, RPS)])


def _sc_hist(dst_chunks, ones16, zeros16):
    f = pl.kernel(
        _hist_body,
        out_type=jax.ShapeDtypeStruct((NC, NPAD, 16), jnp.float32),
        mesh=_mesh(),
        scratch_types=[
            pltpu.VMEM((CPW, CH), jnp.int32),
            pltpu.VMEM((CH, 16), jnp.float32),
            pltpu.VMEM_SHARED((NPAD, 16), jnp.float32),
        ],
    )
    return f(dst_chunks, ones16, zeros16)


def _agg_body(gp_hbm, srcc_hbm, dstc_hbm, z_hbm, out_hbm,
              src_v, dst_v, rows_v, acc, sem):
    c = lax.axis_index("c")
    s = lax.axis_index("s")
    wid = s * NC + c
    r0 = s * RPS

    @pl.when(c == 0)
    def _():
        pltpu.sync_copy(gp_hbm.at[pl.ds(r0, RPS)], acc.at[pl.ds(r0, RPS)])

    @pl.when(c != 0)
    def _():
        pltpu.sync_copy(z_hbm.at[pl.ds(r0, RPS)], acc.at[pl.ds(r0, RPS)])

    base = wid * CPW
    pltpu.sync_copy(srcc_hbm.at[pl.ds(base, CPW)], src_v)
    pltpu.sync_copy(dstc_hbm.at[pl.ds(base, CPW)], dst_v)
    plsc.subcore_barrier()

    def step(j, carry):
        pltpu.async_copy(gp_hbm.at[src_v.at[j]], rows_v, sem).wait()
        pltpu.sync_copy(rows_v, acc.at[dst_v.at[j]], add=True)
        return carry

    lax.fori_loop(0, CPW, step, 0)
    plsc.subcore_barrier()
    pltpu.sync_copy(acc.at[pl.ds(r0, RPS)], out_hbm.at[c, pl.ds(r0, RPS)])


def _sc_agg(gp, src_chunks, dst_chunks, zeros128):
    f = pl.kernel(
        _agg_body,
        out_type=jax.ShapeDtypeStruct((NC, NPAD, C), jnp.float32),
        mesh=_mesh(),
        scratch_types=[
            pltpu.VMEM((CPW, CH), jnp.int32),
            pltpu.VMEM((CPW, CH), jnp.int32),
            pltpu.VMEM((CH, C), jnp.float32),
            pltpu.VMEM_SHARED((NPAD, C), jnp.float32),
            pltpu.SemaphoreType.DMA,
        ],
    )
    return f(gp, src_chunks, dst_chunks, zeros128)


BR = 512


def _prep_body(h0_ref, h1_ref, x_ref, xp_ref, dinvb_ref):
    deg = h0_ref[0, :, :1] + h1_ref[0, :, :1] + 1.0
    db = jnp.broadcast_to(lax.rsqrt(deg), (BR, C))
    dinvb_ref[...] = db
    xp_ref[...] = x_ref[...] * db


def _tc_prep(hist, x_pad):
    grid = NPAD // BR
    return pl.pallas_call(
        _prep_body,
        grid=(grid,),
        in_specs=[
            pl.BlockSpec((1, BR, 16), lambda i: (0, i, 0)),
            pl.BlockSpec((1, BR, 16), lambda i: (1, i, 0)),
            pl.BlockSpec((BR, C), lambda i: (i, 0)),
        ],
        out_specs=[
            pl.BlockSpec((BR, C), lambda i: (i, 0)),
            pl.BlockSpec((BR, C), lambda i: (i, 0)),
        ],
        out_shape=[
            jax.ShapeDtypeStruct((NPAD, C), jnp.float32),
            jax.ShapeDtypeStruct((NPAD, C), jnp.float32),
        ],
    )(hist, hist, x_pad)


def _mid_body(p0_ref, p1_ref, dinvb_ref, W1_ref, b1_ref, W2_ref, tp_ref):
    ax = (p0_ref[0] + p1_ref[0]) * dinvb_ref[...]
    h = jnp.maximum(
        jnp.dot(ax, W1_ref[...], precision=lax.Precision.HIGHEST) + b1_ref[...],
        0.0,
    )
    t = jnp.dot(h, W2_ref[...], precision=lax.Precision.HIGHEST)
    tp_ref[...] = t * dinvb_ref[...]


def _tc_mid(parts, dinvb, W1, b1, W2):
    grid = NPAD // BR
    hw = W1.shape[1]
    return pl.pallas_call(
        _mid_body,
        grid=(grid,),
        in_specs=[
            pl.BlockSpec((1, BR, C), lambda i: (0, i, 0)),
            pl.BlockSpec((1, BR, C), lambda i: (1, i, 0)),
            pl.BlockSpec((BR, C), lambda i: (i, 0)),
            pl.BlockSpec((C, hw), lambda i: (0, 0)),
            pl.BlockSpec((1, hw), lambda i: (0, 0)),
            pl.BlockSpec((hw, C), lambda i: (0, 0)),
        ],
        out_specs=pl.BlockSpec((BR, C), lambda i: (i, 0)),
        out_shape=jax.ShapeDtypeStruct((NPAD, C), jnp.float32),
    )(parts, parts, dinvb, W1, b1.reshape(1, hw), W2)


def _final_body(q0_ref, q1_ref, dinvb_ref, b2_ref, out_ref):
    out_ref[...] = (q0_ref[0] + q1_ref[0]) * dinvb_ref[...] + b2_ref[...]


def _tc_final(parts, dinvb, b2):
    grid = NPAD // BR
    return pl.pallas_call(
        _final_body,
        grid=(grid,),
        in_specs=[
            pl.BlockSpec((1, BR, C), lambda i: (0, i, 0)),
            pl.BlockSpec((1, BR, C), lambda i: (1, i, 0)),
            pl.BlockSpec((BR, C), lambda i: (i, 0)),
            pl.BlockSpec((1, C), lambda i: (0, 0)),
        ],
        out_specs=pl.BlockSpec((BR, C), lambda i: (i, 0)),
        out_shape=jax.ShapeDtypeStruct((N, C), jnp.float32),
    )(parts, parts, dinvb, b2.reshape(1, C))


def kernel(x, edge_index, W1, b1, W2, b2):
    src = edge_index[0]
    dst = edge_index[1]
    padlen = EPAD - E
    srcc = jnp.concatenate(
        [src, jnp.full((padlen,), N, jnp.int32)]).reshape(EPAD // CH, CH)
    dstc = jnp.concatenate(
        [dst, jnp.full((padlen,), N, jnp.int32)]).reshape(EPAD // CH, CH)
    x_pad = jnp.pad(x, ((0, NPAD - N), (0, 0)))
    zeros128 = jnp.zeros((NPAD, C), jnp.float32)
    zeros16 = jnp.zeros((NPAD, 16), jnp.float32)
    ones16 = jnp.ones((CH, 16), jnp.float32)

    hist = _sc_hist(dstc, ones16, zeros16)
    xp, dinvb = _tc_prep(hist, x_pad)
    parts1 = _sc_agg(xp, srcc, dstc, zeros128)
    tp = _tc_mid(parts1, dinvb, W1, b1, W2)
    parts2 = _sc_agg(tp, srcc, dstc, zeros128)
    return _tc_final(parts2, dinvb, b2)


# speedup vs baseline: 8.9627x; 8.9627x over previous
import jax
import jax.numpy as jnp
from jax import lax
from jax.experimental import pallas as pl
from jax.experimental.pallas import tpu as pltpu
from jax.experimental.pallas import tpu_sc as plsc

NC = 2
NS = 16
NW = NC * NS
CH = 128

N = 10000
E = 320000
C = 128
NPAD = 10240
RPS = NPAD // NS
CPW = -(-E // (NW * CH * 8)) * 8
EPAD = NW * CH * CPW


def _mesh():
    return plsc.VectorSubcoreMesh(core_axis_name="c", subcore_axis_name="s")


def _hist_body(dstc_hbm, ones_hbm, z_hbm, out_hbm, dst_v, ones_v, acc):
    c = lax.axis_index("c")
    s = lax.axis_index("s")
    wid = s * NC + c
    r0 = s * RPS
    pltpu.sync_copy(z_hbm.at[pl.ds(r0, RPS)], acc.at[pl.ds(r0, RPS)])
    pltpu.sync_copy(dstc_hbm.at[pl.ds(wid * CPW, CPW)], dst_v)
    pltpu.sync_copy(ones_hbm, ones_v)
    plsc.subcore_barrier()

    def step(j, carry):
        pltpu.sync_copy(ones_v, acc.at[dst_v.at[j]], add=True)
        return carry

    lax.fori_loop(0, CPW, step, 0)
    plsc.subcore_barrier()
    pltpu.sync_copy(acc.at[pl.ds(r0, RPS)], out_hbm.at[c, pl.ds(r0, RPS)])


def _sc_hist(dst_chunks, ones128, zeros128):
    f = pl.kernel(
        _hist_body,
        out_type=jax.ShapeDtypeStruct((NC, NPAD, C), jnp.float32),
        mesh=_mesh(),
        scratch_types=[
            pltpu.VMEM((CPW, CH), jnp.int32),
            pltpu.VMEM((CH, C), jnp.float32),
            pltpu.VMEM_SHARED((NPAD, C), jnp.float32),
        ],
    )
    return f(dst_chunks, ones128, zeros128)


def _agg_body(gp_hbm, srcc_hbm, dstc_hbm, z_hbm, out_hbm,
              src_v, dst_v, rows_v, acc, sem):
    c = lax.axis_index("c")
    s = lax.axis_index("s")
    wid = s * NC + c
    r0 = s * RPS

    @pl.when(c == 0)
    def _():
        pltpu.sync_copy(gp_hbm.at[pl.ds(r0, RPS)], acc.at[pl.ds(r0, RPS)])

    @pl.when(c != 0)
    def _():
        pltpu.sync_copy(z_hbm.at[pl.ds(r0, RPS)], acc.at[pl.ds(r0, RPS)])

    base = wid * CPW
    pltpu.sync_copy(srcc_hbm.at[pl.ds(base, CPW)], src_v)
    pltpu.sync_copy(dstc_hbm.at[pl.ds(base, CPW)], dst_v)
    plsc.subcore_barrier()

    def step(j, carry):
        pltpu.async_copy(gp_hbm.at[src_v.at[j]], rows_v, sem).wait()
        pltpu.sync_copy(rows_v, acc.at[dst_v.at[j]], add=True)
        return carry

    lax.fori_loop(0, CPW, step, 0)
    plsc.subcore_barrier()
    pltpu.sync_copy(acc.at[pl.ds(r0, RPS)], out_hbm.at[c, pl.ds(r0, RPS)])


def _sc_agg(gp, src_chunks, dst_chunks, zeros128):
    f = pl.kernel(
        _agg_body,
        out_type=jax.ShapeDtypeStruct((NC, NPAD, C), jnp.float32),
        mesh=_mesh(),
        scratch_types=[
            pltpu.VMEM((CPW, CH), jnp.int32),
            pltpu.VMEM((CPW, CH), jnp.int32),
            pltpu.VMEM((CH, C), jnp.float32),
            pltpu.VMEM_SHARED((NPAD, C), jnp.float32),
            pltpu.SemaphoreType.DMA,
        ],
    )
    return f(gp, src_chunks, dst_chunks, zeros128)


BR = 512


def _prep_body(h0_ref, h1_ref, x_ref, xp_ref, dinvb_ref):
    deg = h0_ref[0, :, :1] + h1_ref[0, :, :1] + 1.0
    db = jnp.broadcast_to(lax.rsqrt(deg), (BR, C))
    dinvb_ref[...] = db
    xp_ref[...] = x_ref[...] * db


def _tc_prep(hist, x_pad):
    grid = NPAD // BR
    return pl.pallas_call(
        _prep_body,
        grid=(grid,),
        in_specs=[
            pl.BlockSpec((1, BR, C), lambda i: (0, i, 0)),
            pl.BlockSpec((1, BR, C), lambda i: (1, i, 0)),
            pl.BlockSpec((BR, C), lambda i: (i, 0)),
        ],
        out_specs=[
            pl.BlockSpec((BR, C), lambda i: (i, 0)),
            pl.BlockSpec((BR, C), lambda i: (i, 0)),
        ],
        out_shape=[
            jax.ShapeDtypeStruct((NPAD, C), jnp.float32),
            jax.ShapeDtypeStruct((NPAD, C), jnp.float32),
        ],
    )(hist, hist, x_pad)


def _mid_body(p0_ref, p1_ref, dinvb_ref, W1_ref, b1_ref, W2_ref, tp_ref):
    ax = (p0_ref[0] + p1_ref[0]) * dinvb_ref[...]
    h = jnp.maximum(
        jnp.dot(ax, W1_ref[...], precision=lax.Precision.HIGHEST) + b1_ref[...],
        0.0,
    )
    t = jnp.dot(h, W2_ref[...], precision=lax.Precision.HIGHEST)
    tp_ref[...] = t * dinvb_ref[...]


def _tc_mid(parts, dinvb, W1, b1, W2):
    grid = NPAD // BR
    hw = W1.shape[1]
    return pl.pallas_call(
        _mid_body,
        grid=(grid,),
        in_specs=[
            pl.BlockSpec((1, BR, C), lambda i: (0, i, 0)),
            pl.BlockSpec((1, BR, C), lambda i: (1, i, 0)),
            pl.BlockSpec((BR, C), lambda i: (i, 0)),
            pl.BlockSpec((C, hw), lambda i: (0, 0)),
            pl.BlockSpec((1, hw), lambda i: (0, 0)),
            pl.BlockSpec((hw, C), lambda i: (0, 0)),
        ],
        out_specs=pl.BlockSpec((BR, C), lambda i: (i, 0)),
        out_shape=jax.ShapeDtypeStruct((NPAD, C), jnp.float32),
    )(parts, parts, dinvb, W1, b1.reshape(1, hw), W2)


def _final_body(q0_ref, q1_ref, dinvb_ref, b2_ref, out_ref):
    out_ref[...] = (q0_ref[0] + q1_ref[0]) * dinvb_ref[...] + b2_ref[...]


def _tc_final(parts, dinvb, b2):
    grid = NPAD // BR
    return pl.pallas_call(
        _final_body,
        grid=(grid,),
        in_specs=[
            pl.BlockSpec((1, BR, C), lambda i: (0, i, 0)),
            pl.BlockSpec((1, BR, C), lambda i: (1, i, 0)),
            pl.BlockSpec((BR, C), lambda i: (i, 0)),
            pl.BlockSpec((1, C), lambda i: (0, 0)),
        ],
        out_specs=pl.BlockSpec((BR, C), lambda i: (i, 0)),
        out_shape=jax.ShapeDtypeStruct((N, C), jnp.float32),
    )(parts, parts, dinvb, b2.reshape(1, C))


def kernel(x, edge_index, W1, b1, W2, b2):
    src = edge_index[0]
    dst = edge_index[1]
    padlen = EPAD - E
    srcc = jnp.concatenate(
        [src, jnp.full((padlen,), N, jnp.int32)]).reshape(EPAD // CH, CH)
    dstc = jnp.concatenate(
        [dst, jnp.full((padlen,), N, jnp.int32)]).reshape(EPAD // CH, CH)
    x_pad = jnp.pad(x, ((0, NPAD - N), (0, 0)))
    zeros128 = jnp.zeros((NPAD, C), jnp.float32)
    ones128 = jnp.ones((CH, C), jnp.float32)

    hist = _sc_hist(dstc, ones128, zeros128)
    xp, dinvb = _tc_prep(hist, x_pad)
    parts1 = _sc_agg(xp, srcc, dstc, zeros128)
    tp = _tc_mid(parts1, dinvb, W1, b1, W2)
    parts2 = _sc_agg(tp, srcc, dstc, zeros128)
    return _tc_final(parts2, dinvb, b2)


# speedup vs baseline: 9.7417x; 1.0869x over previous
import jax
import jax.numpy as jnp
from jax import lax
from jax.experimental import pallas as pl
from jax.experimental.pallas import tpu as pltpu
from jax.experimental.pallas import tpu_sc as plsc

NC = 2
NS = 16
NW = NC * NS
CH = 128

N = 10000
E = 320000
C = 128
NPAD = 10240
RPS = NPAD // NS
CPW = -(-E // (NW * CH * 8)) * 8
EPAD = NW * CH * CPW


def _mesh():
    return plsc.VectorSubcoreMesh(core_axis_name="c", subcore_axis_name="s")


def _hist_body(dstc_hbm, ones_hbm, z_hbm, out_hbm, dst_v, ones_v, acc):
    c = lax.axis_index("c")
    s = lax.axis_index("s")
    wid = s * NC + c
    r0 = s * RPS
    pltpu.sync_copy(z_hbm.at[pl.ds(r0, RPS)], acc.at[pl.ds(r0, RPS)])
    pltpu.sync_copy(dstc_hbm.at[pl.ds(wid * CPW, CPW)], dst_v)
    pltpu.sync_copy(ones_hbm, ones_v)
    plsc.subcore_barrier()

    def step(j, carry):
        pltpu.sync_copy(ones_v, acc.at[dst_v.at[j]], add=True)
        return carry

    lax.fori_loop(0, CPW, step, 0)
    plsc.subcore_barrier()
    pltpu.sync_copy(acc.at[pl.ds(r0, RPS)], out_hbm.at[c, pl.ds(r0, RPS)])


def _sc_hist(dst_chunks, ones128, zeros128):
    f = pl.kernel(
        _hist_body,
        out_type=jax.ShapeDtypeStruct((NC, NPAD, C), jnp.float32),
        mesh=_mesh(),
        scratch_types=[
            pltpu.VMEM((CPW, CH), jnp.int32),
            pltpu.VMEM((CH, C), jnp.float32),
            pltpu.VMEM_SHARED((NPAD, C), jnp.float32),
        ],
    )
    return f(dst_chunks, ones128, zeros128)


IB = 8
NG = CPW // IB


def _agg_body(gp_hbm, srcc_hbm, dstc_hbm, z_hbm, out_hbm,
              srcA, dstA, srcB, dstB, rows0, rows1, acc,
              isemA, isemB, sem0, sem1):
    c = lax.axis_index("c")
    s = lax.axis_index("s")
    wid = s * NC + c
    r0 = s * RPS

    @pl.when(c == 0)
    def _():
        pltpu.sync_copy(gp_hbm.at[pl.ds(r0, RPS)], acc.at[pl.ds(r0, RPS)])

    @pl.when(c != 0)
    def _():
        pltpu.sync_copy(z_hbm.at[pl.ds(r0, RPS)], acc.at[pl.ds(r0, RPS)])

    base = wid * CPW

    def load_idx(g, sbuf, dbuf, isem):
        pltpu.async_copy(srcc_hbm.at[pl.ds(base + g * IB, IB)], sbuf, isem)
        pltpu.async_copy(dstc_hbm.at[pl.ds(base + g * IB, IB)], dbuf, isem)

    def drain_idx(sbuf, dbuf, isem):
        pltpu.make_async_copy(srcc_hbm.at[pl.ds(base, IB)], sbuf, isem).wait()
        pltpu.make_async_copy(srcc_hbm.at[pl.ds(base, IB)], dbuf, isem).wait()

    def process_group(sbuf, dbuf):
        pltpu.async_copy(gp_hbm.at[sbuf.at[0]], rows0, sem0)
        for k in range(IB):
            cbuf, csem = (rows0, sem0) if k % 2 == 0 else (rows1, sem1)
            nbuf, nsem = (rows1, sem1) if k % 2 == 0 else (rows0, sem0)
            if k + 1 < IB:
                pltpu.async_copy(gp_hbm.at[sbuf.at[k + 1]], nbuf, nsem)
            pltpu.make_async_copy(gp_hbm.at[sbuf.at[0]], cbuf, csem).wait()
            pltpu.sync_copy(cbuf, acc.at[dbuf.at[k]], add=True)

    load_idx(0, srcA, dstA, isemA)
    plsc.subcore_barrier()

    def group_pair(gp, carry):
        g = 2 * gp
        load_idx(g + 1, srcB, dstB, isemB)
        drain_idx(srcA, dstA, isemA)
        process_group(srcA, dstA)

        @pl.when(g + 2 < NG)
        def _():
            load_idx(g + 2, srcA, dstA, isemA)

        drain_idx(srcB, dstB, isemB)
        process_group(srcB, dstB)
        return carry

    lax.fori_loop(0, NG // 2, group_pair, 0)
    plsc.subcore_barrier()
    pltpu.sync_copy(acc.at[pl.ds(r0, RPS)], out_hbm.at[c, pl.ds(r0, RPS)])


def _sc_agg(gp, src_chunks, dst_chunks, zeros128):
    f = pl.kernel(
        _agg_body,
        out_type=jax.ShapeDtypeStruct((NC, NPAD, C), jnp.float32),
        mesh=_mesh(),
        scratch_types=[
            pltpu.VMEM((IB, CH), jnp.int32),
            pltpu.VMEM((IB, CH), jnp.int32),
            pltpu.VMEM((IB, CH), jnp.int32),
            pltpu.VMEM((IB, CH), jnp.int32),
            pltpu.VMEM((CH, C), jnp.float32),
            pltpu.VMEM((CH, C), jnp.float32),
            pltpu.VMEM_SHARED((NPAD, C), jnp.float32),
            pltpu.SemaphoreType.DMA,
            pltpu.SemaphoreType.DMA,
            pltpu.SemaphoreType.DMA,
            pltpu.SemaphoreType.DMA,
        ],
    )
    return f(gp, src_chunks, dst_chunks, zeros128)


BR = 512


def _prep_body(h0_ref, h1_ref, x_ref, xp_ref, dinvb_ref):
    deg = h0_ref[0, :, :1] + h1_ref[0, :, :1] + 1.0
    db = jnp.broadcast_to(lax.rsqrt(deg), (BR, C))
    dinvb_ref[...] = db
    xp_ref[...] = x_ref[...] * db


def _tc_prep(hist, x_pad):
    grid = NPAD // BR
    return pl.pallas_call(
        _prep_body,
        grid=(grid,),
        in_specs=[
            pl.BlockSpec((1, BR, C), lambda i: (0, i, 0)),
            pl.BlockSpec((1, BR, C), lambda i: (1, i, 0)),
            pl.BlockSpec((BR, C), lambda i: (i, 0)),
        ],
        out_specs=[
            pl.BlockSpec((BR, C), lambda i: (i, 0)),
            pl.BlockSpec((BR, C), lambda i: (i, 0)),
        ],
        out_shape=[
            jax.ShapeDtypeStruct((NPAD, C), jnp.float32),
            jax.ShapeDtypeStruct((NPAD, C), jnp.float32),
        ],
    )(hist, hist, x_pad)


def _mid_body(p0_ref, p1_ref, dinvb_ref, W1_ref, b1_ref, W2_ref, tp_ref):
    ax = (p0_ref[0] + p1_ref[0]) * dinvb_ref[...]
    h = jnp.maximum(
        jnp.dot(ax, W1_ref[...], precision=lax.Precision.HIGHEST) + b1_ref[...],
        0.0,
    )
    t = jnp.dot(h, W2_ref[...], precision=lax.Precision.HIGHEST)
    tp_ref[...] = t * dinvb_ref[...]


def _tc_mid(parts, dinvb, W1, b1, W2):
    grid = NPAD // BR
    hw = W1.shape[1]
    return pl.pallas_call(
        _mid_body,
        grid=(grid,),
        in_specs=[
            pl.BlockSpec((1, BR, C), lambda i: (0, i, 0)),
            pl.BlockSpec((1, BR, C), lambda i: (1, i, 0)),
            pl.BlockSpec((BR, C), lambda i: (i, 0)),
            pl.BlockSpec((C, hw), lambda i: (0, 0)),
            pl.BlockSpec((1, hw), lambda i: (0, 0)),
            pl.BlockSpec((hw, C), lambda i: (0, 0)),
        ],
        out_specs=pl.BlockSpec((BR, C), lambda i: (i, 0)),
        out_shape=jax.ShapeDtypeStruct((NPAD, C), jnp.float32),
    )(parts, parts, dinvb, W1, b1.reshape(1, hw), W2)


def _final_body(q0_ref, q1_ref, dinvb_ref, b2_ref, out_ref):
    out_ref[...] = (q0_ref[0] + q1_ref[0]) * dinvb_ref[...] + b2_ref[...]


def _tc_final(parts, dinvb, b2):
    grid = NPAD // BR
    return pl.pallas_call(
        _final_body,
        grid=(grid,),
        in_specs=[
            pl.BlockSpec((1, BR, C), lambda i: (0, i, 0)),
            pl.BlockSpec((1, BR, C), lambda i: (1, i, 0)),
            pl.BlockSpec((BR, C), lambda i: (i, 0)),
            pl.BlockSpec((1, C), lambda i: (0, 0)),
        ],
        out_specs=pl.BlockSpec((BR, C), lambda i: (i, 0)),
        out_shape=jax.ShapeDtypeStruct((N, C), jnp.float32),
    )(parts, parts, dinvb, b2.reshape(1, C))


def kernel(x, edge_index, W1, b1, W2, b2):
    src = edge_index[0]
    dst = edge_index[1]
    padlen = EPAD - E
    srcc = jnp.concatenate(
        [src, jnp.full((padlen,), N, jnp.int32)]).reshape(EPAD // CH, CH)
    dstc = jnp.concatenate(
        [dst, jnp.full((padlen,), N, jnp.int32)]).reshape(EPAD // CH, CH)
    x_pad = jnp.pad(x, ((0, NPAD - N), (0, 0)))
    zeros128 = jnp.zeros((NPAD, C), jnp.float32)
    ones128 = jnp.ones((CH, C), jnp.float32)

    hist = _sc_hist(dstc, ones128, zeros128)
    xp, dinvb = _tc_prep(hist, x_pad)
    parts1 = _sc_agg(xp, srcc, dstc, zeros128)
    tp = _tc_mid(parts1, dinvb, W1, b1, W2)
    parts2 = _sc_agg(tp, srcc, dstc, zeros128)
    return _tc_final(parts2, dinvb, b2)
